# baseline (speedup 1.0000x reference)
import functools

import jax
import jax.numpy as jnp
from jax import lax
from jax.experimental import pallas as pl
from jax.experimental.pallas import tpu as pltpu


_BB = 8


def _main_body(s_ref, x_ref, f_ref, o_ref):
    s = s_ref[0]
    y = x_ref[...] + s * f_ref[...]
    m = jnp.max(y, axis=1, keepdims=True)
    l = jnp.log(jnp.sum(jnp.exp(y - m), axis=1, keepdims=True))
    o_ref[...] = y - m - l


@jax.jit
def _main(s, x, f):
    batch, vocab = x.shape
    return pl.pallas_call(
        _main_body,
        grid=(batch // _BB,),
        in_specs=[
            pl.BlockSpec(memory_space=pltpu.SMEM),
            pl.BlockSpec((_BB, vocab), lambda b: (b, 0)),
            pl.BlockSpec((1, vocab), lambda b: (0, 0)),
        ],
        out_specs=pl.BlockSpec((_BB, vocab), lambda b: (b, 0)),
        out_shape=jax.ShapeDtypeStruct((batch, vocab), jnp.float32),
        compiler_params=pltpu.CompilerParams(
            dimension_semantics=("arbitrary",),
        ),
    )(s, x, f)


def kernel(decoder_output, decoded_words, pattern, w1, lv_table, i, j, batch_size):
    n = jnp.asarray(i, dtype=jnp.int32) * 7 + jnp.asarray(j, dtype=jnp.int32)
    cond = (n > 0) & (jnp.asarray(j) < 7) & (jnp.asarray(i) < 4)

    nd = decoded_words.shape[1]
    idx = jnp.pad(decoded_words[0], (0, 28 - nd))
    prow = jnp.take(pattern, n, axis=0)
    table_flat = lv_table.reshape(-1)

    gathered = jnp.take(table_flat, idx)
    s_scalar = jnp.dot(prow, gathered) * jnp.where(cond, w1[0], 0.0)
    s = jnp.reshape(s_scalar, (1,))

    f = lv_table.reshape(1, -1)
    return _main(s, decoder_output, f)


# speedup vs baseline: 1.0754x; 1.0754x over previous
import functools

import jax
import jax.numpy as jnp
from jax import lax
from jax.experimental import pallas as pl
from jax.experimental.pallas import tpu as pltpu


_BB = 16


def _main_body(s_ref, x_ref, f_ref, o_ref):
    s = s_ref[0]
    y = x_ref[...] + s * f_ref[...]
    m = jnp.max(y, axis=1, keepdims=True)
    l = jnp.log(jnp.sum(jnp.exp(y - m), axis=1, keepdims=True))
    o_ref[...] = y - m - l


@jax.jit
def _main(s, x, f):
    batch, vocab = x.shape
    return pl.pallas_call(
        _main_body,
        grid=(batch // _BB,),
        in_specs=[
            pl.BlockSpec(memory_space=pltpu.SMEM),
            pl.BlockSpec((_BB, vocab), lambda b: (b, 0)),
            pl.BlockSpec((1, vocab), lambda b: (0, 0)),
        ],
        out_specs=pl.BlockSpec((_BB, vocab), lambda b: (b, 0)),
        out_shape=jax.ShapeDtypeStruct((batch, vocab), jnp.float32),
        compiler_params=pltpu.CompilerParams(
            dimension_semantics=("arbitrary",),
        ),
    )(s, x, f)


def kernel(decoder_output, decoded_words, pattern, w1, lv_table, i, j, batch_size):
    n = jnp.asarray(i, dtype=jnp.int32) * 7 + jnp.asarray(j, dtype=jnp.int32)
    cond = (n > 0) & (jnp.asarray(j) < 7) & (jnp.asarray(i) < 4)

    nd = decoded_words.shape[1]
    idx = jnp.pad(decoded_words[0], (0, 28 - nd))
    prow = jnp.take(pattern, n, axis=0)
    table_flat = lv_table.reshape(-1)

    gathered = jnp.take(table_flat, idx)
    s_scalar = jnp.dot(prow, gathered) * jnp.where(cond, w1[0], 0.0)
    s = jnp.reshape(s_scalar, (1,))

    f = lv_table.reshape(1, -1)
    return _main(s, decoder_output, f)
